# baseline (speedup 1.0000x reference)
import functools

import jax
import jax.numpy as jnp
from jax import lax
from jax.experimental import pallas as pl
from jax.experimental.pallas import tpu as pltpu
from jax.experimental.pallas import tpu_sc as plsc

_ETA = 10
_K = 64
_N_ENTS = 1000000
_N_RELS = 1000
_B = 16384
_M = _B * (1 + _ETA)
_NC, _NS = 2, 16
_NW = _NC * _NS
_OPW = _B // _NW
_CH = 128
_NBLK = _OPW // _CH
_PK = 2 * _K


def _tree_sum(vals):
    vals = list(vals)
    while len(vals) > 1:
        nxt = [a + b for a, b in zip(vals[::2], vals[1::2])]
        if len(vals) % 2:
            nxt.append(vals[-1])
        vals = nxt
    return vals[0]


def _sc_scores(sid, spar, rid, rpar, oid, opar, repl, replpar, uvoff,
               ent2, rel2):
    mesh = plsc.VectorSubcoreMesh(
        core_axis_name="c", subcore_axis_name="s",
        num_cores=_NC, num_subcores=_NS)

    @functools.partial(
        pl.kernel,
        mesh=mesh,
        compiler_params=pltpu.CompilerParams(
            needs_layout_passes=False, use_tc_tiling_on_sc=True),
        out_type=jax.ShapeDtypeStruct((_M,), jnp.float32),
        scratch_types=[
            pltpu.VMEM((_OPW,), jnp.int32),
            pltpu.VMEM((_OPW,), jnp.int32),
            pltpu.VMEM((_OPW,), jnp.int32),
            pltpu.VMEM((_OPW,), jnp.int32),
            pltpu.VMEM((_OPW,), jnp.int32),
            pltpu.VMEM((_OPW,), jnp.int32),
            pltpu.VMEM((_ETA * _OPW,), jnp.int32),
            pltpu.VMEM((_ETA * _OPW,), jnp.int32),
            pltpu.VMEM((_ETA * _OPW,), jnp.int32),
            pltpu.VMEM((_CH, _PK), jnp.float32),
            pltpu.VMEM((_CH, _PK), jnp.float32),
            pltpu.VMEM((_CH, _PK), jnp.float32),
            pltpu.VMEM((2, _CH, _PK), jnp.float32),
            pltpu.VMEM((_CH, _PK), jnp.float32),
            pltpu.VMEM((16 * 17,), jnp.float32),
            pltpu.VMEM((_OPW,), jnp.float32),
            pltpu.VMEM((_ETA * _OPW,), jnp.float32),
            pltpu.SemaphoreType.DMA,
            pltpu.SemaphoreType.DMA,
            pltpu.SemaphoreType.DMA,
        ],
    )
    def k(s_hbm, sp_hbm, r_hbm, rp_hbm, o_hbm, op_hbm,
          repl_hbm, replp_hbm, uvo_hbm, ent_hbm, relt_hbm, out_hbm,
          sid_v, spo_v, rid_v, rpo_v, oid_v, opo_v, repl_v, rppo_v, uvo_v,
          sbuf, obuf, pbuf, rbuf, uv_v, pad_v, inp_v, cor_v,
          sem_g, sem_r0, sem_r1):
        wid = lax.axis_index("s") * _NC + lax.axis_index("c")
        wb = wid * _OPW
        pltpu.sync_copy(s_hbm.at[pl.ds(wb, _OPW)], sid_v)
        pltpu.sync_copy(sp_hbm.at[pl.ds(wb, _OPW)], spo_v)
        pltpu.sync_copy(r_hbm.at[pl.ds(wb, _OPW)], rid_v)
        pltpu.sync_copy(rp_hbm.at[pl.ds(wb, _OPW)], rpo_v)
        pltpu.sync_copy(o_hbm.at[pl.ds(wb, _OPW)], oid_v)
        pltpu.sync_copy(op_hbm.at[pl.ds(wb, _OPW)], opo_v)
        for kk in range(_ETA):
            pltpu.sync_copy(
                repl_hbm.at[pl.ds(kk * _B + wb, _OPW)],
                repl_v.at[pl.ds(kk * _OPW, _OPW)])
            pltpu.sync_copy(
                replp_hbm.at[pl.ds(kk * _B + wb, _OPW)],
                rppo_v.at[pl.ds(kk * _OPW, _OPW)])
            pltpu.sync_copy(
                uvo_hbm.at[pl.ds(kk * _B + wb, _OPW)],
                uvo_v.at[pl.ds(kk * _OPW, _OPW)])

        lane = lax.iota(jnp.int32, 16)
        rsems = (sem_r0, sem_r1)

        def start_orig(blk):
            off = blk * _CH
            pltpu.async_copy(
                ent_hbm.at[sid_v.at[pl.ds(off, _CH)]], sbuf, sem_g)
            pltpu.async_copy(
                ent_hbm.at[oid_v.at[pl.ds(off, _CH)]], obuf, sem_g)
            pltpu.async_copy(
                relt_hbm.at[rid_v.at[pl.ds(off, _CH)]], pbuf, sem_g)

        def wait_orig():
            pltpu.make_async_copy(
                ent_hbm.at[sid_v.at[pl.ds(0, _CH)]], sbuf, sem_g).wait()
            pltpu.make_async_copy(
                ent_hbm.at[oid_v.at[pl.ds(0, _CH)]], obuf, sem_g).wait()
            pltpu.make_async_copy(
                relt_hbm.at[rid_v.at[pl.ds(0, _CH)]], pbuf, sem_g).wait()

        def start_repl(blk, kk, b):
            pltpu.async_copy(
                ent_hbm.at[repl_v.at[pl.ds(kk * _OPW + blk * _CH, _CH)]],
                rbuf.at[b], rsems[b])

        def wait_repl(b):
            pltpu.make_async_copy(
                ent_hbm.at[repl_v.at[pl.ds(0, _CH)]],
                rbuf.at[b], rsems[b]).wait()

        def transpose_sum():
            cols = []
            for d in range(16):
                cols.append(plsc.load_gather(pad_v, [lane * 17 + d]))
            return _tree_sum(cols)

        def block_body(blk):
            boff = blk * _CH
            wait_orig()
            start_repl(blk, 0, 0)
            start_repl(blk, 1, 1)

            def uv_group(g):
                grow = g * 16
                spvec = spo_v[pl.ds(boff + grow, 16)]
                opvec = opo_v[pl.ds(boff + grow, 16)]
                rpvec = rpo_v[pl.ds(boff + grow, 16)]
                for r in range(16):
                    so = spvec[r]
                    po = rpvec[r]
                    oo = opvec[r]
                    terms = []
                    for cc in range(_K // 16):
                        sv = sbuf[grow + r, pl.ds(so + cc * 16, 16)]
                        pv = pbuf[grow + r, pl.ds(po + cc * 16, 16)]
                        ov = obuf[grow + r, pl.ds(oo + cc * 16, 16)]
                        u = sv * pv
                        v = pv * ov
                        uv_v[grow + r, pl.ds(cc * 16, 16)] = u
                        uv_v[grow + r, pl.ds(_K + cc * 16, 16)] = v
                        terms.append(u * ov)
                    pad_v[pl.ds(r * 17, 16)] = _tree_sum(terms)
                inp_v[pl.ds(boff + grow, 16)] = transpose_sum()

            pl.loop(0, _CH // 16)(uv_group)

            @pl.when(blk + 1 < _NBLK)
            def _():
                start_orig(blk + 1)

            def corr_pair(kk0):
                for b in range(2):
                    kk = kk0 + b
                    wait_repl(b)
                    rb = rbuf.at[b]

                    def corr_group(g):
                        grow = g * 16
                        kbase = kk * _OPW + boff + grow
                        uvvec = uvo_v[pl.ds(kbase, 16)]
                        rpvec = rppo_v[pl.ds(kbase, 16)]
                        for r in range(16):
                            uvo = uvvec[r]
                            rpo = rpvec[r]
                            terms = []
                            for cc in range(_K // 16):
                                w = uv_v[grow + r, pl.ds(uvo + cc * 16, 16)]
                                rv = rb[grow + r, pl.ds(rpo + cc * 16, 16)]
                                terms.append(w * rv)
                            pad_v[pl.ds(r * 17, 16)] = _tree_sum(terms)
                        cor_v[pl.ds(kbase, 16)] = transpose_sum()

                    pl.loop(0, _CH // 16)(corr_group)

                    @pl.when(kk + 2 < _ETA)
                    def _():
                        start_repl(blk, kk + 2, b)

            pl.loop(0, _ETA, step=2)(corr_pair)

        start_orig(0)
        pl.loop(0, _NBLK)(block_body)

        pltpu.sync_copy(inp_v, out_hbm.at[pl.ds(wb, _OPW)])
        for kk in range(_ETA):
            pltpu.sync_copy(
                cor_v.at[pl.ds(kk * _OPW, _OPW)],
                out_hbm.at[pl.ds((kk + 1) * _B + wb, _OPW)])

    return k(sid, spar, rid, rpar, oid, opar, repl, replpar, uvoff,
             ent2, rel2)


def kernel(triples, ent_emb, rel_emb):
    t = triples.astype(jnp.int32)
    key = jax.random.key(42)
    m = _B * _ETA
    k1, k2 = jax.random.split(key)
    keep_subj = jax.random.randint(k1, (m,), 0, 2)
    repl = jax.random.randint(k2, (m,), 0, _N_ENTS)

    subj = t[:, 0]
    rel = t[:, 1]
    obj = t[:, 2]
    ent2 = ent_emb.reshape(_N_ENTS // 2, _PK)
    rel2 = rel_emb.reshape(_N_RELS // 2, _PK)

    scores = _sc_scores(
        subj // 2, (subj % 2) * _K,
        rel // 2, (rel % 2) * _K,
        obj // 2, (obj % 2) * _K,
        repl // 2, (repl % 2) * _K,
        (1 - keep_subj) * _K,
        ent2, rel2)
    return scores[:_B], scores[_B:]


# speedup vs baseline: 1.0285x; 1.0285x over previous
import functools

import jax
import jax.numpy as jnp
from jax import lax
from jax.experimental import pallas as pl
from jax.experimental.pallas import tpu as pltpu
from jax.experimental.pallas import tpu_sc as plsc

_ETA = 10
_K = 64
_N_ENTS = 1000000
_N_RELS = 1000
_B = 16384
_M = _B * (1 + _ETA)
_NC, _NS = 2, 16
_NW = _NC * _NS
_OPW = _B // _NW
_CH = 128
_NBLK = _OPW // _CH
_NRING = 4


def _tree_sum(vals):
    vals = list(vals)
    while len(vals) > 1:
        nxt = [a + b for a, b in zip(vals[::2], vals[1::2])]
        if len(vals) % 2:
            nxt.append(vals[-1])
        vals = nxt
    return vals[0]


def _sc_scores(subj2, rel2, obj2, repl2, keep2, ent_emb, rel_emb):
    mesh = plsc.VectorSubcoreMesh(
        core_axis_name="c", subcore_axis_name="s",
        num_cores=_NC, num_subcores=_NS)

    @functools.partial(
        pl.kernel,
        mesh=mesh,
        compiler_params=pltpu.CompilerParams(
            needs_layout_passes=False, use_tc_tiling_on_sc=False),
        out_type=jax.ShapeDtypeStruct((_M,), jnp.float32),
        scratch_types=[
            pltpu.VMEM((_OPW,), jnp.int32),
            pltpu.VMEM((_OPW,), jnp.int32),
            pltpu.VMEM((_OPW,), jnp.int32),
            pltpu.VMEM((_ETA, _OPW), jnp.int32),
            pltpu.VMEM((_ETA, _OPW), jnp.int32),
            pltpu.VMEM((_CH, _K), jnp.float32),
            pltpu.VMEM((_CH, _K), jnp.float32),
            pltpu.VMEM((_CH, _K), jnp.float32),
            pltpu.VMEM((_NRING, _CH, _K), jnp.float32),
            pltpu.VMEM((_CH, 2 * _K), jnp.float32),
            pltpu.VMEM((16, 17), jnp.float32),
            pltpu.VMEM((_OPW,), jnp.float32),
            pltpu.VMEM((_ETA, _OPW), jnp.float32),
            pltpu.SemaphoreType.DMA,
            pltpu.SemaphoreType.DMA,
            pltpu.SemaphoreType.DMA,
            pltpu.SemaphoreType.DMA,
            pltpu.SemaphoreType.DMA,
        ],
    )
    def k(s_hbm, r_hbm, o_hbm, repl_hbm, keep_hbm, ent_hbm, relt_hbm, out_hbm,
          sid_v, rid_v, oid_v, repl_v, keep_v,
          sbuf, obuf, pbuf, rbuf, uv_v, pad_v, inp_v, cor_v,
          sem_g, sem_r0, sem_r1, sem_r2, sem_r3):
        wid = lax.axis_index("s") * _NC + lax.axis_index("c")
        wb = wid * _OPW
        pltpu.sync_copy(s_hbm.at[wid], sid_v)
        pltpu.sync_copy(r_hbm.at[wid], rid_v)
        pltpu.sync_copy(o_hbm.at[wid], oid_v)
        pltpu.sync_copy(repl_hbm.at[:, pl.ds(wb, _OPW)], repl_v)
        pltpu.sync_copy(keep_hbm.at[:, pl.ds(wb, _OPW)], keep_v)

        lane = lax.iota(jnp.int32, 16)
        rsems = (sem_r0, sem_r1, sem_r2, sem_r3)

        def start_orig(blk):
            off = blk * _CH
            pltpu.async_copy(
                ent_hbm.at[sid_v.at[pl.ds(off, _CH)]], sbuf, sem_g)
            pltpu.async_copy(
                ent_hbm.at[oid_v.at[pl.ds(off, _CH)]], obuf, sem_g)
            pltpu.async_copy(
                relt_hbm.at[rid_v.at[pl.ds(off, _CH)]], pbuf, sem_g)

        def wait_orig():
            pltpu.make_async_copy(
                ent_hbm.at[sid_v.at[pl.ds(0, _CH)]], sbuf, sem_g).wait()
            pltpu.make_async_copy(
                ent_hbm.at[oid_v.at[pl.ds(0, _CH)]], obuf, sem_g).wait()
            pltpu.make_async_copy(
                relt_hbm.at[rid_v.at[pl.ds(0, _CH)]], pbuf, sem_g).wait()

        def start_repl(blk, kk, b):
            pltpu.async_copy(
                ent_hbm.at[repl_v.at[kk, pl.ds(blk * _CH, _CH)]],
                rbuf.at[b], rsems[b])

        def wait_repl(b):
            pltpu.make_async_copy(
                ent_hbm.at[repl_v.at[0, pl.ds(0, _CH)]],
                rbuf.at[b], rsems[b]).wait()

        def transpose_sum():
            cols = []
            for d in range(16):
                dsplat = jnp.full((16,), d, jnp.int32)
                cols.append(plsc.load_gather(pad_v, [lane, dsplat]))
            return _tree_sum(cols)

        def block_body(blk):
            boff = blk * _CH
            wait_orig()
            for b in range(_NRING):
                start_repl(blk, b, b)

            def uv_group(g):
                grow = g * 16
                for r in range(16):
                    terms = []
                    for cc in range(_K // 16):
                        sv = sbuf[grow + r, pl.ds(cc * 16, 16)]
                        pv = pbuf[grow + r, pl.ds(cc * 16, 16)]
                        ov = obuf[grow + r, pl.ds(cc * 16, 16)]
                        u = sv * pv
                        v = pv * ov
                        uv_v[grow + r, pl.ds(cc * 16, 16)] = u
                        uv_v[grow + r, pl.ds(_K + cc * 16, 16)] = v
                        terms.append(u * ov)
                    pad_v[r, pl.ds(0, 16)] = _tree_sum(terms)
                inp_v[pl.ds(boff + grow, 16)] = transpose_sum()

            pl.loop(0, _CH // 16)(uv_group)

            @pl.when(blk + 1 < _NBLK)
            def _():
                start_orig(blk + 1)

            for kk in range(_ETA):
                b = kk % _NRING
                wait_repl(b)
                rb = rbuf.at[b]

                def corr_group(g, kk=kk, rb=rb):
                    grow = g * 16
                    kvec = keep_v[kk, pl.ds(boff + grow, 16)]
                    offv = (1 - kvec) * _K
                    for r in range(16):
                        off_r = offv[r]
                        terms = []
                        for cc in range(_K // 16):
                            w = uv_v[grow + r, pl.ds(off_r + cc * 16, 16)]
                            rv = rb[grow + r, pl.ds(cc * 16, 16)]
                            terms.append(w * rv)
                        pad_v[r, pl.ds(0, 16)] = _tree_sum(terms)
                    cor_v[kk, pl.ds(boff + grow, 16)] = transpose_sum()

                pl.loop(0, _CH // 16)(corr_group)

                if kk + _NRING < _ETA:
                    start_repl(blk, kk + _NRING, b)

        start_orig(0)
        pl.loop(0, _NBLK)(block_body)

        pltpu.sync_copy(inp_v, out_hbm.at[pl.ds(wb, _OPW)])
        for kk in range(_ETA):
            pltpu.sync_copy(
                cor_v.at[kk],
                out_hbm.at[pl.ds((kk + 1) * _B + wb, _OPW)])

    return k(subj2, rel2, obj2, repl2, keep2, ent_emb, rel_emb)


def kernel(triples, ent_emb, rel_emb):
    t = triples.astype(jnp.int32)
    key = jax.random.key(42)
    m = _B * _ETA
    k1, k2 = jax.random.split(key)
    keep_subj = jax.random.randint(k1, (m,), 0, 2)
    repl = jax.random.randint(k2, (m,), 0, _N_ENTS)

    subj2 = t[:, 0].reshape(_NW, _OPW)
    rel2 = t[:, 1].reshape(_NW, _OPW)
    obj2 = t[:, 2].reshape(_NW, _OPW)
    repl2 = repl.reshape(_ETA, _B)
    keep2 = keep_subj.reshape(_ETA, _B)

    scores = _sc_scores(subj2, rel2, obj2, repl2, keep2, ent_emb, rel_emb)
    return scores[:_B], scores[_B:]


# speedup vs baseline: 1.0342x; 1.0056x over previous
import functools

import jax
import jax.numpy as jnp
from jax import lax
from jax.experimental import pallas as pl
from jax.experimental.pallas import tpu as pltpu
from jax.experimental.pallas import tpu_sc as plsc

_ETA = 10
_K = 64
_N_ENTS = 1000000
_N_RELS = 1000
_B = 16384
_M = _B * (1 + _ETA)
_NC, _NS = 2, 16
_NW = _NC * _NS
_OPW = _B // _NW
_CH = 128
_NBLK = _OPW // _CH


def _tree_sum(vals):
    vals = list(vals)
    while len(vals) > 1:
        nxt = [a + b for a, b in zip(vals[::2], vals[1::2])]
        if len(vals) % 2:
            nxt.append(vals[-1])
        vals = nxt
    return vals[0]


def _sc_scores(subj2, rel2, obj2, repl3, keep3, ent_emb, rel_emb):
    mesh = plsc.VectorSubcoreMesh(
        core_axis_name="c", subcore_axis_name="s",
        num_cores=_NC, num_subcores=_NS)

    @functools.partial(
        pl.kernel,
        mesh=mesh,
        compiler_params=pltpu.CompilerParams(
            needs_layout_passes=False, use_tc_tiling_on_sc=False),
        out_type=jax.ShapeDtypeStruct((_M,), jnp.float32),
        scratch_types=[
            pltpu.VMEM((_OPW,), jnp.int32),
            pltpu.VMEM((_OPW,), jnp.int32),
            pltpu.VMEM((_OPW,), jnp.int32),
            pltpu.VMEM((_ETA, _OPW), jnp.int32),
            pltpu.VMEM((_ETA, _OPW), jnp.int32),
            pltpu.VMEM((_CH, _K), jnp.float32),
            pltpu.VMEM((_CH, _K), jnp.float32),
            pltpu.VMEM((_CH, _K), jnp.float32),
            pltpu.VMEM((2, _CH, _K), jnp.float32),
            pltpu.VMEM((_CH, 2 * _K), jnp.float32),
            pltpu.VMEM((16, 17), jnp.float32),
            pltpu.VMEM((_OPW,), jnp.float32),
            pltpu.VMEM((_ETA, _OPW), jnp.float32),
            pltpu.SemaphoreType.DMA,
            pltpu.SemaphoreType.DMA,
            pltpu.SemaphoreType.DMA,
        ],
    )
    def k(s_hbm, r_hbm, o_hbm, repl_hbm, keep_hbm, ent_hbm, relt_hbm, out_hbm,
          sid_v, rid_v, oid_v, repl_v, keep_v,
          sbuf, obuf, pbuf, rbuf, uv_v, pad_v, inp_v, cor_v,
          sem_g, sem_r0, sem_r1):
        wid = lax.axis_index("s") * _NC + lax.axis_index("c")
        pltpu.sync_copy(s_hbm.at[wid], sid_v)
        pltpu.sync_copy(r_hbm.at[wid], rid_v)
        pltpu.sync_copy(o_hbm.at[wid], oid_v)
        pltpu.sync_copy(repl_hbm.at[wid], repl_v)
        pltpu.sync_copy(keep_hbm.at[wid], keep_v)

        lane = lax.iota(jnp.int32, 16)
        rsems = (sem_r0, sem_r1)

        def start_orig(blk):
            off = blk * _CH
            pltpu.async_copy(
                ent_hbm.at[sid_v.at[pl.ds(off, _CH)]], sbuf, sem_g)
            pltpu.async_copy(
                ent_hbm.at[oid_v.at[pl.ds(off, _CH)]], obuf, sem_g)
            pltpu.async_copy(
                relt_hbm.at[rid_v.at[pl.ds(off, _CH)]], pbuf, sem_g)

        def wait_orig():
            pltpu.make_async_copy(
                ent_hbm.at[sid_v.at[pl.ds(0, _CH)]], sbuf, sem_g).wait()
            pltpu.make_async_copy(
                ent_hbm.at[oid_v.at[pl.ds(0, _CH)]], obuf, sem_g).wait()
            pltpu.make_async_copy(
                relt_hbm.at[rid_v.at[pl.ds(0, _CH)]], pbuf, sem_g).wait()

        def start_repl(blk, kk, b):
            pltpu.async_copy(
                ent_hbm.at[repl_v.at[kk, pl.ds(blk * _CH, _CH)]],
                rbuf.at[b], rsems[b])

        def wait_repl(b):
            pltpu.make_async_copy(
                ent_hbm.at[repl_v.at[0, pl.ds(0, _CH)]],
                rbuf.at[b], rsems[b]).wait()

        def transpose_sum(cols_src):
            cols = []
            for d in range(16):
                dsplat = jnp.full((16,), d, jnp.int32)
                cols.append(plsc.load_gather(cols_src, [lane, dsplat]))
            return _tree_sum(cols)

        def block_body(blk):
            boff = blk * _CH
            wait_orig()
            start_repl(blk, 0, 0)
            start_repl(blk, 1, 1)

            def uv_group(g):
                grow = g * 16
                for r in range(16):
                    terms = []
                    for cc in range(_K // 16):
                        sv = sbuf[grow + r, pl.ds(cc * 16, 16)]
                        pv = pbuf[grow + r, pl.ds(cc * 16, 16)]
                        ov = obuf[grow + r, pl.ds(cc * 16, 16)]
                        u = sv * pv
                        v = pv * ov
                        uv_v[grow + r, pl.ds(cc * 16, 16)] = u
                        uv_v[grow + r, pl.ds(_K + cc * 16, 16)] = v
                        terms.append(u * ov)
                    pad_v[r, pl.ds(0, 16)] = _tree_sum(terms)
                inp_v[pl.ds(boff + grow, 16)] = transpose_sum(pad_v)

            pl.loop(0, _CH // 16)(uv_group)

            @pl.when(blk + 1 < _NBLK)
            def _():
                start_orig(blk + 1)

            def corr_pair(kk0):
                for b in range(2):
                    kk = kk0 + b
                    wait_repl(b)
                    rb = rbuf.at[b]

                    def corr_group(g):
                        grow = g * 16
                        kvec = keep_v[kk, pl.ds(boff + grow, 16)]
                        offv = (1 - kvec) * _K
                        for r in range(16):
                            off_r = offv[r]
                            terms = []
                            for cc in range(_K // 16):
                                w = uv_v[grow + r, pl.ds(off_r + cc * 16, 16)]
                                rv = rb[grow + r, pl.ds(cc * 16, 16)]
                                terms.append(w * rv)
                            pad_v[r, pl.ds(0, 16)] = _tree_sum(terms)
                        cor_v[kk, pl.ds(boff + grow, 16)] = transpose_sum(pad_v)

                    pl.loop(0, _CH // 16)(corr_group)

                    @pl.when(kk + 2 < _ETA)
                    def _():
                        start_repl(blk, kk + 2, b)

            pl.loop(0, _ETA, step=2)(corr_pair)

        start_orig(0)
        pl.loop(0, _NBLK)(block_body)

        pltpu.sync_copy(inp_v, out_hbm.at[pl.ds(wid * _OPW, _OPW)])
        for kk in range(_ETA):
            pltpu.sync_copy(
                cor_v.at[kk],
                out_hbm.at[pl.ds((kk + 1) * _B + wid * _OPW, _OPW)])

    return k(subj2, rel2, obj2, repl3, keep3, ent_emb, rel_emb)


def kernel(triples, ent_emb, rel_emb):
    t = triples.astype(jnp.int32)
    key = jax.random.key(42)
    m = _B * _ETA
    k1, k2 = jax.random.split(key)
    keep_subj = jax.random.randint(k1, (m,), 0, 2)
    repl = jax.random.randint(k2, (m,), 0, _N_ENTS)

    subj2 = t[:, 0].reshape(_NW, _OPW)
    rel2 = t[:, 1].reshape(_NW, _OPW)
    obj2 = t[:, 2].reshape(_NW, _OPW)
    repl3 = repl.reshape(_ETA, _NW, _OPW).transpose(1, 0, 2)
    keep3 = keep_subj.reshape(_ETA, _NW, _OPW).transpose(1, 0, 2)

    scores = _sc_scores(subj2, rel2, obj2, repl3, keep3, ent_emb, rel_emb)
    return scores[:_B], scores[_B:]
